# baseline (speedup 1.0000x reference)
import functools

import jax
import jax.numpy as jnp
from jax import lax
from jax.experimental import pallas as pl
from jax.experimental.pallas import tpu as pltpu
from jax.experimental.pallas import tpu_sc as plsc

_NF = 26
_D = 64
_B = 16384
_NC, _NS = 2, 16
_NW = _NC * _NS
_BPW = _B // _NW

_mesh = plsc.VectorSubcoreMesh(core_axis_name="c", subcore_axis_name="s")


@functools.partial(
    pl.kernel,
    out_type=jax.ShapeDtypeStruct((_B, _NF * _D), jnp.float32),
    mesh=_mesh,
    compiler_params=pltpu.CompilerParams(use_tc_tiling_on_sc=False),
    scratch_types=[
        pltpu.VMEM((_NF, _BPW), jnp.int32),
        pltpu.VMEM((2, _BPW, _D), jnp.float32),
        pltpu.SemaphoreType.DMA,
        pltpu.SemaphoreType.DMA,
    ],
)
def _embed_kernel(xT, *rest):
    tables = rest[:_NF]
    out = rest[_NF]
    idx_v, rows_v, sem0, sem1 = rest[_NF + 1:]
    sems = (sem0, sem1)

    wid = lax.axis_index("s") * _NC + lax.axis_index("c")
    base = wid * _BPW

    pltpu.sync_copy(xT.at[:, pl.ds(base, _BPW)], idx_v)

    copies = [None, None]
    copies[0] = pltpu.async_copy(tables[0].at[idx_v.at[0]], rows_v.at[0], sems[0])
    for f in range(_NF):
        b = f % 2
        if f + 1 < _NF:
            nb = (f + 1) % 2
            copies[nb] = pltpu.async_copy(
                tables[f + 1].at[idx_v.at[f + 1]], rows_v.at[nb], sems[nb])
        copies[b].wait()
        pltpu.sync_copy(rows_v.at[b],
                        out.at[pl.ds(base, _BPW), pl.ds(f * _D, _D)])


def kernel(x, table_0, table_1, table_2, table_3, table_4, table_5,
           table_6, table_7, table_8, table_9, table_10, table_11,
           table_12, table_13, table_14, table_15, table_16, table_17,
           table_18, table_19, table_20, table_21, table_22, table_23,
           table_24, table_25):
    xT = x.T
    return _embed_kernel(
        xT, table_0, table_1, table_2, table_3, table_4, table_5,
        table_6, table_7, table_8, table_9, table_10, table_11,
        table_12, table_13, table_14, table_15, table_16, table_17,
        table_18, table_19, table_20, table_21, table_22, table_23,
        table_24, table_25)
